# baseline (speedup 1.0000x reference)
import jax
import jax.numpy as jnp
from jax import lax
from jax.experimental import pallas as pl
from jax.experimental.pallas import tpu as pltpu
from jax.experimental.pallas import tpu_sc as plsc

BATCH = 16384
FACTORS = 64

_info = plsc.get_sparse_core_info()
_NC, _NS, _L = _info.num_cores, _info.num_subcores, _info.num_lanes
_NW = _NC * _NS
_BPW = BATCH // _NW
_GROUPS = _BPW // _L


def _fm_body(user_hbm, item_hbm, eu_hbm, ei_hbm, ub_hbm, ib_hbm, b_hbm,
             out_hbm, uidx_v, iidx_v, urows_v, irows_v, ub_v, ib_v,
             bias_v, out_v, sem):
    wid = lax.axis_index("s") * _NC + lax.axis_index("c")
    base = wid * _BPW

    pltpu.sync_copy(user_hbm.at[pl.ds(base, _BPW)], uidx_v)
    pltpu.sync_copy(item_hbm.at[pl.ds(base, _BPW)], iidx_v)
    pltpu.sync_copy(b_hbm, bias_v)

    cp_u = pltpu.async_copy(eu_hbm.at[uidx_v], urows_v, sem)
    cp_i = pltpu.async_copy(ei_hbm.at[iidx_v], irows_v, sem)
    cp_ub = pltpu.async_copy(ub_hbm.at[uidx_v], ub_v, sem)
    cp_ib = pltpu.async_copy(ib_hbm.at[iidx_v], ib_v, sem)
    cp_u.wait()
    cp_i.wait()
    cp_ub.wait()
    cp_ib.wait()

    bias = bias_v[...]

    def group(g, carry):
        row = g * _L + lax.iota(jnp.int32, _L)
        acc = bias
        acc = acc + ub_v[pl.ds(g * _L, _L)]
        acc = acc + ib_v[pl.ds(g * _L, _L)]
        for f in range(FACTORS):
            col = jnp.full((_L,), f, jnp.int32)
            u = plsc.load_gather(urows_v, [row, col])
            v = plsc.load_gather(irows_v, [row, col])
            acc = acc + u * v
        out_v[pl.ds(g * _L, _L)] = acc
        return carry

    lax.fori_loop(0, _GROUPS, group, 0)
    pltpu.sync_copy(out_v, out_hbm.at[pl.ds(base, _BPW)])


def kernel(user, item, embed_user, embed_item, u_bias, i_bias, bias_):
    mesh = plsc.VectorSubcoreMesh(core_axis_name="c", subcore_axis_name="s")
    fm = pl.kernel(
        _fm_body,
        out_type=jax.ShapeDtypeStruct((BATCH,), jnp.float32),
        mesh=mesh,
        compiler_params=pltpu.CompilerParams(
            needs_layout_passes=False, use_tc_tiling_on_sc=False),
        scratch_types=[
            pltpu.VMEM((_BPW,), jnp.int32),
            pltpu.VMEM((_BPW,), jnp.int32),
            pltpu.VMEM((_BPW, FACTORS), jnp.float32),
            pltpu.VMEM((_BPW, FACTORS), jnp.float32),
            pltpu.VMEM((_BPW,), jnp.float32),
            pltpu.VMEM((_BPW,), jnp.float32),
            pltpu.VMEM((_L,), jnp.float32),
            pltpu.VMEM((_BPW,), jnp.float32),
            pltpu.SemaphoreType.DMA,
        ],
    )
    return fm(user.astype(jnp.int32), item.astype(jnp.int32),
              embed_user, embed_item,
              u_bias.reshape(-1), i_bias.reshape(-1),
              jnp.broadcast_to(bias_, (_L,)))
